# baseline (speedup 1.0000x reference)
import functools

import jax
import jax.numpy as jnp
from jax import lax
from jax.experimental import pallas as pl
from jax.experimental.pallas import tpu as pltpu
from jax.experimental.pallas import tpu_sc as plsc

VOCAB = 1000000
EMBED_DIM = 64
BATCH = 16384
HIST = 50

_NC = 2
_NS = 16
_NW = _NC * _NS
_B = BATCH * HIST
_PER_W = _B // _NW
_CHUNK = 128
_NCHUNK = _PER_W // _CHUNK
_NBUF = 4
_PADD = 128


def _make_gather():
    mesh = plsc.VectorSubcoreMesh(core_axis_name="c", subcore_axis_name="s")

    @functools.partial(
        pl.kernel,
        mesh=mesh,
        out_type=jax.ShapeDtypeStruct((_B, _PADD), jnp.float32),
        scratch_types=(
            [pltpu.VMEM((_NCHUNK, _CHUNK), jnp.int32)]
            + [pltpu.VMEM((_CHUNK, _PADD), jnp.float32) for _ in range(_NBUF)]
            + [pltpu.SemaphoreType.DMA for _ in range(2 * _NBUF)]
        ),
    )
    def gather_kernel(idx_hbm, table_hbm, out_hbm, idx_v, *bufs_and_sems):
        rows = bufs_and_sems[:_NBUF]
        gsem = bufs_and_sems[_NBUF:2 * _NBUF]
        osem = bufs_and_sems[2 * _NBUF:]
        wid = lax.axis_index("s") * _NC + lax.axis_index("c")
        pltpu.sync_copy(idx_hbm.at[wid], idx_v)
        base = wid * _PER_W

        def gather_chunk(k, b):
            pltpu.async_copy(table_hbm.at[idx_v.at[k]], rows[b], gsem[b])

        for b in range(_NBUF):
            gather_chunk(b, b)

        def group(j, carry):
            for b in range(_NBUF):
                k = j + b
                pltpu.make_async_copy(
                    table_hbm.at[pl.ds(0, _CHUNK)], rows[b], gsem[b]
                ).wait()
                pltpu.async_copy(
                    rows[b],
                    out_hbm.at[pl.ds(base + k * _CHUNK, _CHUNK)],
                    osem[b],
                )

                @pl.when(k + _NBUF < _NCHUNK)
                def _():
                    pltpu.make_async_copy(
                        rows[b], out_hbm.at[pl.ds(0, _CHUNK)], osem[b]
                    ).wait()
                    gather_chunk(k + _NBUF, b)

            return carry

        lax.fori_loop(0, _NCHUNK // _NBUF, lambda i, c: group(i * _NBUF, c), 0,
                      unroll=False)

        for b in range(_NBUF):
            pltpu.make_async_copy(
                rows[b], out_hbm.at[pl.ds(0, _CHUNK)], osem[b]
            ).wait()

    return gather_kernel


_gather = _make_gather()


def kernel(nodes_batch, features):
    idx = nodes_batch.reshape(_NW, _NCHUNK, _CHUNK).astype(jnp.int32)
    table = jnp.pad(features, ((0, 0), (0, _PADD - EMBED_DIM)))
    out = _gather(idx, table)
    return out[:, :EMBED_DIM].reshape(BATCH, HIST, EMBED_DIM)


# speedup vs baseline: 1.1511x; 1.1511x over previous
import functools

import jax
import jax.numpy as jnp
from jax import lax
from jax.experimental import pallas as pl
from jax.experimental.pallas import tpu as pltpu
from jax.experimental.pallas import tpu_sc as plsc

VOCAB = 1000000
EMBED_DIM = 64
BATCH = 16384
HIST = 50

_NC = 2
_NS = 16
_NW = _NC * _NS
_BPW = BATCH // _NW
_CHUNK = 128
_NCB = _BPW // _CHUNK
_PADD = 128


def _make_gather():
    mesh = plsc.VectorSubcoreMesh(core_axis_name="c", subcore_axis_name="s")

    @functools.partial(
        pl.kernel,
        mesh=mesh,
        out_type=jax.ShapeDtypeStruct((HIST, BATCH, _PADD), jnp.float32),
        scratch_types=(
            [pltpu.VMEM((HIST, _NCB, _CHUNK), jnp.int32)]
            + [pltpu.VMEM((_CHUNK, _PADD), jnp.float32) for _ in range(_NCB)]
            + [pltpu.SemaphoreType.DMA for _ in range(2 * _NCB)]
        ),
    )
    def gather_kernel(idx_hbm, table_hbm, out_hbm, idx_v, *bufs_and_sems):
        rows = bufs_and_sems[:_NCB]
        gsem = bufs_and_sems[_NCB:2 * _NCB]
        osem = bufs_and_sems[2 * _NCB:]
        wid = lax.axis_index("s") * _NC + lax.axis_index("c")
        pltpu.sync_copy(idx_hbm.at[wid], idx_v)
        base = wid * _BPW

        def gather_chunk(h, b):
            pltpu.async_copy(table_hbm.at[idx_v.at[h, b]], rows[b], gsem[b])

        for b in range(_NCB):
            gather_chunk(0, b)

        def per_hist(h, carry):
            for b in range(_NCB):
                pltpu.make_async_copy(
                    table_hbm.at[pl.ds(0, _CHUNK)], rows[b], gsem[b]
                ).wait()
                pltpu.async_copy(
                    rows[b],
                    out_hbm.at[h, pl.ds(base + b * _CHUNK, _CHUNK)],
                    osem[b],
                )

                @pl.when(h + 1 < HIST)
                def _():
                    pltpu.make_async_copy(
                        rows[b], out_hbm.at[0, pl.ds(0, _CHUNK)], osem[b]
                    ).wait()
                    gather_chunk(h + 1, b)

            return carry

        lax.fori_loop(0, HIST, per_hist, 0, unroll=False)

        for b in range(_NCB):
            pltpu.make_async_copy(
                rows[b], out_hbm.at[0, pl.ds(0, _CHUNK)], osem[b]
            ).wait()

    return gather_kernel


_gather = _make_gather()


def kernel(nodes_batch, features):
    idx = (
        nodes_batch.astype(jnp.int32)
        .T.reshape(HIST, _NW, _NCB * _CHUNK)
        .transpose(1, 0, 2)
        .reshape(_NW, HIST, _NCB, _CHUNK)
    )
    table = jnp.pad(features, ((0, 0), (0, _PADD - EMBED_DIM)))
    out = _gather(idx, table)
    return out[:, :, :EMBED_DIM].transpose(1, 0, 2)
